# baseline (speedup 1.0000x reference)
import functools

import jax
import jax.numpy as jnp
from jax import lax
from jax.experimental import pallas as pl
from jax.experimental.pallas import tpu as pltpu
from jax.experimental.pallas import tpu_sc as plsc

N_NODES = 10000
N_EDGES = 320000
K = 4
D = 128

NC = 2
NS = 16
L = 16
NW = NC * NS
EPW = N_EDGES // NW
CHUNK = 2000
NCHUNK = EPW // CHUNK
GROUPS = CHUNK // L

BN = 1024
BE = 32768


def _tables_body(fs_ref, fd_ref, ei_ref, al_ref, ar_ref,
                 el_ref, er_ref, src_ref, dst_ref):
    ones = jnp.full((1, D), 1.0, jnp.float32)
    dn = (((1,), (1,)), ((), ()))
    xs = (fs_ref[...] * al_ref[...]).reshape(BN * K, D)
    xd = (fd_ref[...] * ar_ref[...]).reshape(BN * K, D)
    el_ref[...] = lax.dot_general(ones, xs, dn,
                                  preferred_element_type=jnp.float32)[0]
    er_ref[...] = lax.dot_general(ones, xd, dn,
                                  preferred_element_type=jnp.float32)[0]
    src_ref[...] = ei_ref[0]
    dst_ref[...] = ei_ref[1]


def _compute_tables(feat_src, feat_dst, edge_index, attn_l, attn_r):
    NB = (N_NODES + BN - 1) // BN
    return pl.pallas_call(
        _tables_body,
        grid=(NB,),
        in_specs=[
            pl.BlockSpec((BN, K, D), lambda i: (i, 0, 0)),
            pl.BlockSpec((BN, K, D), lambda i: (i, 0, 0)),
            pl.BlockSpec((2, BE), lambda i: (0, i)),
            pl.BlockSpec((K, D), lambda i: (0, 0)),
            pl.BlockSpec((K, D), lambda i: (0, 0)),
        ],
        out_specs=[
            pl.BlockSpec((BN * K,), lambda i: (i,)),
            pl.BlockSpec((BN * K,), lambda i: (i,)),
            pl.BlockSpec((BE,), lambda i: (i,)),
            pl.BlockSpec((BE,), lambda i: (i,)),
        ],
        out_shape=[
            jax.ShapeDtypeStruct((N_NODES * K,), jnp.float32),
            jax.ShapeDtypeStruct((N_NODES * K,), jnp.float32),
            jax.ShapeDtypeStruct((N_EDGES,), jnp.int32),
            jax.ShapeDtypeStruct((N_EDGES,), jnp.int32),
        ],
    )(feat_src, feat_dst, edge_index, attn_l[0], attn_r[0])


def _gather_body(el_hbm, er_hbm, src_hbm, dst_hbm, out_hbm,
                 el_v, er_v, sidx0, sidx1, didx0, didx1, out0, out1,
                 sem_tab, sem_idx0, sem_idx1, sem_out0, sem_out1):
    sidx_b = (sidx0, sidx1)
    didx_b = (didx0, didx1)
    out_b = (out0, out1)
    sem_idx_b = (sem_idx0, sem_idx1)
    sem_out_b = (sem_out0, sem_out1)
    cid = lax.axis_index("c")
    sid = lax.axis_index("s")
    wid = sid * NC + cid
    base = wid * EPW

    tab_l = pltpu.async_copy(el_hbm, el_v, sem_tab)
    tab_r = pltpu.async_copy(er_hbm, er_v, sem_tab)

    def fire_idx(c):
        cb = base + c * CHUNK
        b = c % 2
        return (pltpu.async_copy(src_hbm.at[pl.ds(cb, CHUNK)],
                                 sidx_b[b], sem_idx_b[b]),
                pltpu.async_copy(dst_hbm.at[pl.ds(cb, CHUNK)],
                                 didx_b[b], sem_idx_b[b]))

    idx_cp = fire_idx(0)
    tab_l.wait()
    tab_r.wait()

    out_cp = [None, None]
    for c in range(NCHUNK):
        b = c % 2
        nxt = fire_idx(c + 1) if c + 1 < NCHUNK else None
        idx_cp[0].wait()
        idx_cp[1].wait()
        if out_cp[b] is not None:
            for cp in out_cp[b]:
                cp.wait()

        sidx = sidx_b[b]
        didx = didx_b[b]
        outb = out_b[b]

        @plsc.parallel_loop(0, GROUPS, 1, unroll=8)
        def group(g):
            off = g * L
            sb = sidx[pl.ds(off, L)] * K
            db = didx[pl.ds(off, L)] * K
            for k in range(K):
                a = plsc.load_gather(el_v, [sb + k])
                b_ = plsc.load_gather(er_v, [db + k])
                outb[pl.ds(k * CHUNK + off, L)] = a + b_

        cb = base + c * CHUNK
        out_cp[b] = [pltpu.async_copy(outb.at[pl.ds(k * CHUNK, CHUNK)],
                                      out_hbm.at[pl.ds(k * N_EDGES + cb, CHUNK)],
                                      sem_out_b[b])
                     for k in range(K)]
        idx_cp = nxt

    for b in range(2):
        if out_cp[b] is not None:
            for cp in out_cp[b]:
                cp.wait()


_gather_call = functools.partial(
    pl.kernel,
    out_type=jax.ShapeDtypeStruct((N_EDGES * K,), jnp.float32),
    mesh=plsc.VectorSubcoreMesh(core_axis_name="c", subcore_axis_name="s"),
    compiler_params=pltpu.CompilerParams(needs_layout_passes=False),
    scratch_types=[
        pltpu.VMEM((N_NODES * K,), jnp.float32),
        pltpu.VMEM((N_NODES * K,), jnp.float32),
        pltpu.VMEM((CHUNK,), jnp.int32),
        pltpu.VMEM((CHUNK,), jnp.int32),
        pltpu.VMEM((CHUNK,), jnp.int32),
        pltpu.VMEM((CHUNK,), jnp.int32),
        pltpu.VMEM((K * CHUNK,), jnp.float32),
        pltpu.VMEM((K * CHUNK,), jnp.float32),
        pltpu.SemaphoreType.DMA,
        pltpu.SemaphoreType.DMA,
        pltpu.SemaphoreType.DMA,
        pltpu.SemaphoreType.DMA,
        pltpu.SemaphoreType.DMA,
    ],
)(_gather_body)


def kernel(feat_src, feat_dst, edge_index, attn_l, attn_r):
    el, er, src, dst = _compute_tables(
        feat_src, feat_dst, edge_index.astype(jnp.int32), attn_l, attn_r)
    flat = _gather_call(el, er, src, dst)
    return jnp.transpose(flat.reshape(K, 1, N_EDGES), (2, 0, 1))


# speedup vs baseline: 1.0515x; 1.0515x over previous
import functools

import jax
import jax.numpy as jnp
from jax import lax
from jax.experimental import pallas as pl
from jax.experimental.pallas import tpu as pltpu
from jax.experimental.pallas import tpu_sc as plsc

N_NODES = 10000
N_EDGES = 320000
K = 4
D = 128

NC = 2
NS = 16
L = 16
NW = NC * NS
EPW = N_EDGES // NW
CHUNK = 2000
NCHUNK = EPW // CHUNK
GROUPS = CHUNK // L

BN = 2048
BE = 65536


def _tables_body(fs_ref, fd_ref, ei_ref, al_ref, ar_ref,
                 el_ref, er_ref, src_ref, dst_ref):
    ones = jnp.full((1, D), 1.0, jnp.float32)
    dn = (((1,), (1,)), ((), ()))
    xs = (fs_ref[...] * al_ref[...]).reshape(BN * K, D)
    xd = (fd_ref[...] * ar_ref[...]).reshape(BN * K, D)
    el_ref[...] = lax.dot_general(ones, xs, dn,
                                  preferred_element_type=jnp.float32)[0]
    er_ref[...] = lax.dot_general(ones, xd, dn,
                                  preferred_element_type=jnp.float32)[0]
    src_ref[...] = ei_ref[0]
    dst_ref[...] = ei_ref[1]


def _compute_tables(feat_src, feat_dst, edge_index, attn_l, attn_r):
    NB = (N_NODES + BN - 1) // BN
    return pl.pallas_call(
        _tables_body,
        grid=(NB,),
        in_specs=[
            pl.BlockSpec((BN, K, D), lambda i: (i, 0, 0)),
            pl.BlockSpec((BN, K, D), lambda i: (i, 0, 0)),
            pl.BlockSpec((2, BE), lambda i: (0, i)),
            pl.BlockSpec((K, D), lambda i: (0, 0)),
            pl.BlockSpec((K, D), lambda i: (0, 0)),
        ],
        out_specs=[
            pl.BlockSpec((BN * K,), lambda i: (i,)),
            pl.BlockSpec((BN * K,), lambda i: (i,)),
            pl.BlockSpec((BE,), lambda i: (i,)),
            pl.BlockSpec((BE,), lambda i: (i,)),
        ],
        out_shape=[
            jax.ShapeDtypeStruct((N_NODES * K,), jnp.float32),
            jax.ShapeDtypeStruct((N_NODES * K,), jnp.float32),
            jax.ShapeDtypeStruct((N_EDGES,), jnp.int32),
            jax.ShapeDtypeStruct((N_EDGES,), jnp.int32),
        ],
    )(feat_src, feat_dst, edge_index, attn_l[0], attn_r[0])


def _gather_body(el_hbm, er_hbm, src_hbm, dst_hbm, out_hbm,
                 el_v, er_v, sidx0, sidx1, didx0, didx1, out0, out1,
                 sem_tab, sem_idx0, sem_idx1, sem_out0, sem_out1):
    sidx_b = (sidx0, sidx1)
    didx_b = (didx0, didx1)
    out_b = (out0, out1)
    sem_idx_b = (sem_idx0, sem_idx1)
    sem_out_b = (sem_out0, sem_out1)
    cid = lax.axis_index("c")
    sid = lax.axis_index("s")
    wid = sid * NC + cid
    base = wid * EPW

    tab_l = pltpu.async_copy(el_hbm, el_v, sem_tab)
    tab_r = pltpu.async_copy(er_hbm, er_v, sem_tab)

    def fire_idx(c):
        cb = base + c * CHUNK
        b = c % 2
        return (pltpu.async_copy(src_hbm.at[pl.ds(cb, CHUNK)],
                                 sidx_b[b], sem_idx_b[b]),
                pltpu.async_copy(dst_hbm.at[pl.ds(cb, CHUNK)],
                                 didx_b[b], sem_idx_b[b]))

    idx_cp = fire_idx(0)
    tab_l.wait()
    tab_r.wait()

    out_cp = [None, None]
    for c in range(NCHUNK):
        b = c % 2
        nxt = fire_idx(c + 1) if c + 1 < NCHUNK else None
        idx_cp[0].wait()
        idx_cp[1].wait()
        if out_cp[b] is not None:
            for cp in out_cp[b]:
                cp.wait()

        sidx = sidx_b[b]
        didx = didx_b[b]
        outb = out_b[b]

        @plsc.parallel_loop(0, GROUPS, 1, unroll=4)
        def group(g):
            off = g * L
            sb = sidx[pl.ds(off, L)] * K
            db = didx[pl.ds(off, L)] * K
            for k in range(K):
                a = plsc.load_gather(el_v, [sb + k])
                b_ = plsc.load_gather(er_v, [db + k])
                outb[pl.ds(k * CHUNK + off, L)] = a + b_

        cb = base + c * CHUNK
        out_cp[b] = [pltpu.async_copy(outb.at[pl.ds(k * CHUNK, CHUNK)],
                                      out_hbm.at[pl.ds(k * N_EDGES + cb, CHUNK)],
                                      sem_out_b[b])
                     for k in range(K)]
        idx_cp = nxt

    for b in range(2):
        if out_cp[b] is not None:
            for cp in out_cp[b]:
                cp.wait()


_gather_call = functools.partial(
    pl.kernel,
    out_type=jax.ShapeDtypeStruct((N_EDGES * K,), jnp.float32),
    mesh=plsc.VectorSubcoreMesh(core_axis_name="c", subcore_axis_name="s"),
    compiler_params=pltpu.CompilerParams(needs_layout_passes=False),
    scratch_types=[
        pltpu.VMEM((N_NODES * K,), jnp.float32),
        pltpu.VMEM((N_NODES * K,), jnp.float32),
        pltpu.VMEM((CHUNK,), jnp.int32),
        pltpu.VMEM((CHUNK,), jnp.int32),
        pltpu.VMEM((CHUNK,), jnp.int32),
        pltpu.VMEM((CHUNK,), jnp.int32),
        pltpu.VMEM((K * CHUNK,), jnp.float32),
        pltpu.VMEM((K * CHUNK,), jnp.float32),
        pltpu.SemaphoreType.DMA,
        pltpu.SemaphoreType.DMA,
        pltpu.SemaphoreType.DMA,
        pltpu.SemaphoreType.DMA,
        pltpu.SemaphoreType.DMA,
    ],
)(_gather_body)


def kernel(feat_src, feat_dst, edge_index, attn_l, attn_r):
    el, er, src, dst = _compute_tables(
        feat_src, feat_dst, edge_index.astype(jnp.int32), attn_l, attn_r)
    flat = _gather_call(el, er, src, dst)
    return jnp.transpose(flat.reshape(K, 1, N_EDGES), (2, 0, 1))
